# baseline (speedup 1.0000x reference)
import functools

import jax
import jax.numpy as jnp
from jax import lax
from jax.experimental import pallas as pl
from jax.experimental.pallas import tpu as pltpu
from jax.experimental.pallas import tpu_sc as plsc

AH = 7
AW = 7
SCALE = 0.125

_NW = 32
_BLK = 32


def _prep_body(H, W, rois_ref, idx_ref, w_ref):
    rois = rois_ref[...]
    Rb = rois.shape[0]
    j = lax.broadcasted_iota(jnp.int32, (Rb, 4 * AH * AW), 1)
    binq = j // 4
    k = j % 4
    ph = (binq // AW).astype(jnp.float32)
    pw = (binq % AW).astype(jnp.float32)
    dh = k // 2
    dw = k % 2
    batch = rois[:, 0:1].astype(jnp.int32)
    x1 = rois[:, 1:2] * SCALE
    y1 = rois[:, 2:3] * SCALE
    x2 = rois[:, 3:4] * SCALE
    y2 = rois[:, 4:5] * SCALE
    bin_h = jnp.maximum(y2 - y1, 0.0) / float(AH - 1)
    bin_w = jnp.maximum(x2 - x1, 0.0) / float(AW - 1)
    h = y1 + ph * bin_h
    w = x1 + pw * bin_w
    valid = (h >= 0.0) & (h < H) & (w >= 0.0) & (w < W)
    hs = jnp.clip(jnp.floor(h), 0.0, float(H - 2))
    ws = jnp.clip(jnp.floor(w), 0.0, float(W - 2))
    hr = h - hs
    wr = w - ws
    wgt_h = jnp.where(dh == 0, 1.0 - hr, hr)
    wgt_w = jnp.where(dw == 0, 1.0 - wr, wr)
    w_ref[...] = jnp.where(valid, wgt_h * wgt_w, 0.0)
    idx_ref[...] = (batch * (H * W)
                    + (hs.astype(jnp.int32) + dh) * W
                    + ws.astype(jnp.int32) + dw)


def _sc_body(n_blocks, idx4_hbm, w4_hbm, fmap_hbm, out_hbm,
             idx_v, w_v, corners0, corners1, out0, out1,
             gsem0, gsem1, osem0, osem1):
    wid = lax.axis_index("s") * 2 + lax.axis_index("c")
    base = wid * (n_blocks * _BLK)

    pltpu.sync_copy(idx4_hbm.at[pl.ds(base * 4, n_blocks * 4 * _BLK)], idx_v)
    pltpu.sync_copy(w4_hbm.at[pl.ds(base * 4, n_blocks * 4 * _BLK)], w_v)

    corners = (corners0, corners1)
    outs = (out0, out1)
    gsems = (gsem0, gsem1)
    osems = (osem0, osem1)

    def gather_copy(b, par):
        return pltpu.make_async_copy(
            fmap_hbm.at[idx_v.at[pl.ds(b * 4 * _BLK, 4 * _BLK)]],
            corners[par], gsems[par])

    def out_copy(b, par):
        start = base + b * _BLK
        return pltpu.make_async_copy(
            outs[par], out_hbm.at[pl.ds(start, _BLK)], osems[par])

    gather_copy(0, 0).start()

    def blk2_body(b2, carry):
        for par in (0, 1):
            b = 2 * b2 + par
            nxt = 1 - par

            @pl.when(b + 1 < n_blocks)
            def _():
                gather_copy(b + 1, nxt).start()

            gather_copy(b, par).wait()

            @pl.when(b2 >= 1)
            def _():
                out_copy(b - 2, par).wait()

            cv = corners[par]
            ov = outs[par]

            def g_body(g, c2):
                wv = w_v[pl.ds(b * 4 * _BLK + 16 * g, 16)]
                for t in range(4):
                    s = 4 * g + t
                    w00 = wv[4 * t]
                    w01 = wv[4 * t + 1]
                    w10 = wv[4 * t + 2]
                    w11 = wv[4 * t + 3]
                    for c in range(16):
                        sl = pl.ds(c * 16, 16)
                        ov[s, sl] = (w00 * cv[4 * s, sl]
                                     + w01 * cv[4 * s + 1, sl]
                                     + w10 * cv[4 * s + 2, sl]
                                     + w11 * cv[4 * s + 3, sl])
                return c2

            lax.fori_loop(0, _BLK // 4, g_body, 0)
            out_copy(b, par).start()
        return carry

    lax.fori_loop(0, n_blocks // 2, blk2_body, 0)
    out_copy(n_blocks - 2, 0).wait()
    out_copy(n_blocks - 1, 1).wait()


def _nchw_body(in_ref, out_ref):
    hdim, wdim = in_ref.shape[2], in_ref.shape[3]
    eye = (lax.broadcasted_iota(jnp.int32, (wdim, wdim), 0)
           == lax.broadcasted_iota(jnp.int32, (wdim, wdim), 1)
           ).astype(jnp.float32)

    def h_body(h, carry):
        out_ref[0, h] = lax.dot_general(
            eye, in_ref[0, :, h, :], (((1,), (1,)), ((), ())),
            preferred_element_type=jnp.float32)
        return carry

    lax.fori_loop(0, hdim, h_body, 0)


def _trans_body(in_ref, out_ref):
    x = in_ref[...]
    tb = out_ref.shape[0]
    nbins = AH * AW
    x = x.reshape(tb, nbins, x.shape[-1])
    eye = (lax.broadcasted_iota(jnp.int32, (nbins, nbins), 0)
           == lax.broadcasted_iota(jnp.int32, (nbins, nbins), 1)
           ).astype(jnp.float32)
    out_ref[...] = lax.dot_general(
        x, eye, (((1,), (0,)), ((), ())),
        preferred_element_type=jnp.float32)


@functools.lru_cache(maxsize=None)
def _build(N, C, H, W, R):
    chunk = _NW * _BLK
    r_pad = R
    while (r_pad * AH * AW) % chunk != 0:
        r_pad += 1
    s_pad = r_pad * AH * AW
    n_blocks = s_pad // (_NW * _BLK)

    rb = 256
    while r_pad % rb != 0:
        rb //= 2
    grid = (r_pad // rb,)
    prep = pl.pallas_call(
        functools.partial(_prep_body, H, W),
        grid=grid,
        in_specs=[pl.BlockSpec((rb, 5), lambda i: (i, 0))],
        out_specs=[pl.BlockSpec((rb, 4 * AH * AW), lambda i: (i, 0)),
                   pl.BlockSpec((rb, 4 * AH * AW), lambda i: (i, 0))],
        out_shape=[jax.ShapeDtypeStruct((r_pad, 4 * AH * AW), jnp.int32),
                   jax.ShapeDtypeStruct((r_pad, 4 * AH * AW), jnp.float32)],
    )

    mesh = plsc.VectorSubcoreMesh(core_axis_name="c", subcore_axis_name="s")
    sc = functools.partial(
        pl.kernel,
        mesh=mesh,
        out_type=jax.ShapeDtypeStruct((s_pad, C), jnp.float32),
        scratch_types=[
            pltpu.VMEM((n_blocks * 4 * _BLK,), jnp.int32),
            pltpu.VMEM((n_blocks * 4 * _BLK,), jnp.float32),
            pltpu.VMEM((4 * _BLK, C), jnp.float32),
            pltpu.VMEM((4 * _BLK, C), jnp.float32),
            pltpu.VMEM((_BLK, C), jnp.float32),
            pltpu.VMEM((_BLK, C), jnp.float32),
            pltpu.SemaphoreType.DMA,
            pltpu.SemaphoreType.DMA,
            pltpu.SemaphoreType.DMA,
            pltpu.SemaphoreType.DMA,
        ],
    )(functools.partial(_sc_body, n_blocks))

    tb = 16
    nbins = AH * AW
    trans = pl.pallas_call(
        _trans_body,
        grid=(R // tb,),
        in_specs=[pl.BlockSpec((tb * nbins, C), lambda i: (i, 0))],
        out_specs=pl.BlockSpec((tb, C, nbins), lambda i: (i, 0, 0)),
        out_shape=jax.ShapeDtypeStruct((R, C, nbins), jnp.float32),
    )

    nchw = pl.pallas_call(
        _nchw_body,
        grid=(N,),
        in_specs=[pl.BlockSpec((1, C, H, W), lambda n: (n, 0, 0, 0))],
        out_specs=pl.BlockSpec((1, H, W, C), lambda n: (n, 0, 0, 0)),
        out_shape=jax.ShapeDtypeStruct((N, H, W, C), jnp.float32),
    )

    return prep, sc, trans, nchw, r_pad, s_pad


def kernel(features, rois):
    N, C, H, W = features.shape
    R = rois.shape[0]
    prep, sc, trans, nchw, r_pad, s_pad = _build(N, C, H, W, R)
    fmap = nchw(features).reshape(N * H * W, C)
    rois_p = jnp.pad(rois, ((0, r_pad - R), (0, 0)))
    idx4, w4 = prep(rois_p)
    out = sc(idx4.reshape(-1), w4.reshape(-1), fmap)
    out = trans(out)
    return out.reshape(R, C, AH, AW)


# speedup vs baseline: 1.1141x; 1.1141x over previous
import functools

import jax
import jax.numpy as jnp
from jax import lax
from jax.experimental import pallas as pl
from jax.experimental.pallas import tpu as pltpu
from jax.experimental.pallas import tpu_sc as plsc

AH = 7
AW = 7
SCALE = 0.125

_NW = 32
_BLK = 32


def _prep_body(H, W, rois_ref, idx_ref, w_ref):
    rois = rois_ref[...]
    Rb = rois.shape[0]
    j = lax.broadcasted_iota(jnp.int32, (Rb, 4 * AH * AW), 1)
    binq = j // 4
    k = j % 4
    ph = (binq // AW).astype(jnp.float32)
    pw = (binq % AW).astype(jnp.float32)
    dh = k // 2
    dw = k % 2
    batch = rois[:, 0:1].astype(jnp.int32)
    x1 = rois[:, 1:2] * SCALE
    y1 = rois[:, 2:3] * SCALE
    x2 = rois[:, 3:4] * SCALE
    y2 = rois[:, 4:5] * SCALE
    bin_h = jnp.maximum(y2 - y1, 0.0) / float(AH - 1)
    bin_w = jnp.maximum(x2 - x1, 0.0) / float(AW - 1)
    h = y1 + ph * bin_h
    w = x1 + pw * bin_w
    valid = (h >= 0.0) & (h < H) & (w >= 0.0) & (w < W)
    hs = jnp.clip(jnp.floor(h), 0.0, float(H - 2))
    ws = jnp.clip(jnp.floor(w), 0.0, float(W - 2))
    hr = h - hs
    wr = w - ws
    wgt_h = jnp.where(dh == 0, 1.0 - hr, hr)
    wgt_w = jnp.where(dw == 0, 1.0 - wr, wr)
    w_ref[...] = jnp.where(valid, wgt_h * wgt_w, 0.0)
    idx_ref[...] = (batch * (H * W)
                    + (hs.astype(jnp.int32) + dh) * W
                    + ws.astype(jnp.int32) + dw)


def _sc_body(n_blocks, idx4_hbm, w4_hbm, fmap_hbm, out_hbm,
             idx_v, w_v, corners0, corners1, out0, out1,
             gsem0, gsem1, osem0, osem1):
    wid = lax.axis_index("s") * 2 + lax.axis_index("c")
    base = wid * (n_blocks * _BLK)

    pltpu.sync_copy(idx4_hbm.at[pl.ds(base * 4, n_blocks * 4 * _BLK)], idx_v)
    pltpu.sync_copy(w4_hbm.at[pl.ds(base * 4, n_blocks * 4 * _BLK)], w_v)

    corners = (corners0, corners1)
    outs = (out0, out1)
    gsems = (gsem0, gsem1)
    osems = (osem0, osem1)

    def gather_copy(b, par):
        return pltpu.make_async_copy(
            fmap_hbm.at[idx_v.at[pl.ds(b * 4 * _BLK, 4 * _BLK)]],
            corners[par], gsems[par])

    def out_copy(b, par):
        start = base + b * _BLK
        return pltpu.make_async_copy(
            outs[par], out_hbm.at[pl.ds(start, _BLK)], osems[par])

    gather_copy(0, 0).start()

    def blk2_body(b2, carry):
        for par in (0, 1):
            b = 2 * b2 + par
            nxt = 1 - par

            @pl.when(b + 1 < n_blocks)
            def _():
                gather_copy(b + 1, nxt).start()

            gather_copy(b, par).wait()

            @pl.when(b2 >= 1)
            def _():
                out_copy(b - 2, par).wait()

            cv = corners[par]
            ov = outs[par]

            def g_body(g, c2):
                wv = w_v[pl.ds(b * 4 * _BLK + 16 * g, 16)]
                for t in range(4):
                    s = 4 * g + t
                    w00 = wv[4 * t]
                    w01 = wv[4 * t + 1]
                    w10 = wv[4 * t + 2]
                    w11 = wv[4 * t + 3]
                    for c in range(16):
                        sl = pl.ds(c * 16, 16)
                        ov[s, sl] = (w00 * cv[4 * s, sl]
                                     + w01 * cv[4 * s + 1, sl]
                                     + w10 * cv[4 * s + 2, sl]
                                     + w11 * cv[4 * s + 3, sl])
                return c2

            lax.fori_loop(0, _BLK // 4, g_body, 0)
            out_copy(b, par).start()
        return carry

    lax.fori_loop(0, n_blocks // 2, blk2_body, 0)
    out_copy(n_blocks - 2, 0).wait()
    out_copy(n_blocks - 1, 1).wait()


def _nchw_body(in_ref, out_ref):
    hdim, wdim = in_ref.shape[2], in_ref.shape[3]
    eye = (lax.broadcasted_iota(jnp.int32, (wdim, wdim), 0)
           == lax.broadcasted_iota(jnp.int32, (wdim, wdim), 1)
           ).astype(jnp.float32)

    def h_body(h, carry):
        out_ref[0, h] = lax.dot_general(
            eye, in_ref[0, :, h, :], (((1,), (1,)), ((), ())),
            preferred_element_type=jnp.float32)
        return carry

    lax.fori_loop(0, hdim, h_body, 0)


def _trans_body(in_ref, out_ref):
    x = in_ref[...]
    tb = out_ref.shape[0]
    nbins = AH * AW
    x = x.reshape(tb, nbins, x.shape[-1])
    eye = (lax.broadcasted_iota(jnp.int32, (nbins, nbins), 0)
           == lax.broadcasted_iota(jnp.int32, (nbins, nbins), 1)
           ).astype(jnp.float32)
    out_ref[...] = lax.dot_general(
        x, eye, (((1,), (0,)), ((), ())),
        preferred_element_type=jnp.float32)


@functools.lru_cache(maxsize=None)
def _build(N, C, H, W, R):
    chunk = _NW * _BLK
    r_pad = R
    while (r_pad * AH * AW) % chunk != 0:
        r_pad += 1
    s_pad = r_pad * AH * AW
    n_blocks = s_pad // (_NW * _BLK)

    rb = 256
    while r_pad % rb != 0:
        rb //= 2
    grid = (r_pad // rb,)
    prep = pl.pallas_call(
        functools.partial(_prep_body, H, W),
        grid=grid,
        in_specs=[pl.BlockSpec((rb, 5), lambda i: (i, 0))],
        out_specs=[pl.BlockSpec((rb, 4 * AH * AW), lambda i: (i, 0)),
                   pl.BlockSpec((rb, 4 * AH * AW), lambda i: (i, 0))],
        out_shape=[jax.ShapeDtypeStruct((r_pad, 4 * AH * AW), jnp.int32),
                   jax.ShapeDtypeStruct((r_pad, 4 * AH * AW), jnp.float32)],
    )

    mesh = plsc.VectorSubcoreMesh(core_axis_name="c", subcore_axis_name="s")
    sc = functools.partial(
        pl.kernel,
        mesh=mesh,
        out_type=jax.ShapeDtypeStruct((s_pad, C), jnp.float32),
        scratch_types=[
            pltpu.VMEM((n_blocks * 4 * _BLK,), jnp.int32),
            pltpu.VMEM((n_blocks * 4 * _BLK,), jnp.float32),
            pltpu.VMEM((4 * _BLK, C), jnp.float32),
            pltpu.VMEM((4 * _BLK, C), jnp.float32),
            pltpu.VMEM((_BLK, C), jnp.float32),
            pltpu.VMEM((_BLK, C), jnp.float32),
            pltpu.SemaphoreType.DMA,
            pltpu.SemaphoreType.DMA,
            pltpu.SemaphoreType.DMA,
            pltpu.SemaphoreType.DMA,
        ],
    )(functools.partial(_sc_body, n_blocks))

    tb = 16
    nbins = AH * AW
    trans = pl.pallas_call(
        _trans_body,
        grid=(R // tb,),
        in_specs=[pl.BlockSpec((tb * nbins, C), lambda i: (i, 0))],
        out_specs=pl.BlockSpec((tb, C, nbins), lambda i: (i, 0, 0)),
        out_shape=jax.ShapeDtypeStruct((R, C, nbins), jnp.float32),
    )

    nchw = pl.pallas_call(
        _nchw_body,
        grid=(N,),
        in_specs=[pl.BlockSpec((1, C, H, W), lambda n: (n, 0, 0, 0))],
        out_specs=pl.BlockSpec((1, H, W, C), lambda n: (n, 0, 0, 0)),
        out_shape=jax.ShapeDtypeStruct((N, H, W, C), jnp.float32),
    )

    return prep, sc, trans, nchw, r_pad, s_pad


def kernel(features, rois):
    N, C, H, W = features.shape
    R = rois.shape[0]
    prep, sc, trans, nchw, r_pad, s_pad = _build(N, C, H, W, R)
    del nchw
    fmap = jnp.transpose(features, (0, 2, 3, 1)).reshape(N * H * W, C)
    rois_p = jnp.pad(rois, ((0, r_pad - R), (0, 0)))
    idx4, w4 = prep(rois_p)
    out = sc(idx4.reshape(-1), w4.reshape(-1), fmap)
    out = trans(out)
    return out.reshape(R, C, AH, AW)
